# Initial kernel scaffold; baseline (speedup 1.0000x reference)
import jax
import jax.numpy as jnp
from jax.experimental import pallas as pl


def kernel(x, token_table, pos_table):
    raise NotImplementedError("write your pallas kernel here")


# speedup vs baseline: 4.2628x; 4.2628x over previous
import functools

import jax
import jax.numpy as jnp
from jax import lax
from jax.experimental import pallas as pl
from jax.experimental.pallas import tpu as pltpu
from jax.experimental.pallas import tpu_sc as plsc

_B, _L, _V, _D = 1024, 200, 100000, 128
_NW = 32
_SEQ_PER_W = _B // _NW
_HALF = 100
_LANES = 16


def _emb_body(idx_hbm, tok_hbm, pos_hbm, out_hbm, idx_v, pos_v, buf, gsem, ssem):
    wid = lax.axis_index("s") * 2 + lax.axis_index("c")
    pltpu.sync_copy(idx_hbm.at[pl.ds(64 * wid, 64)], idx_v)
    pltpu.sync_copy(pos_hbm, pos_v)

    def chunk(c, carry):
        g0 = pltpu.async_copy(
            tok_hbm.at[idx_v.at[2 * c]], buf.at[pl.ds(0, _HALF)], gsem)
        g1 = pltpu.async_copy(
            tok_hbm.at[idx_v.at[2 * c + 1]], buf.at[pl.ds(_HALF, _HALF)], gsem)
        g0.wait()
        g1.wait()

        def row(r, rc):
            for j in range(_D // _LANES):
                sl = pl.ds(_LANES * j, _LANES)
                plsc.addupdate(buf.at[r, sl], pos_v[r, sl])
            return rc

        lax.fori_loop(0, _L, row, 0)
        pltpu.async_copy(buf, out_hbm.at[_SEQ_PER_W * wid + c], ssem).wait()
        return carry

    lax.fori_loop(0, _SEQ_PER_W, chunk, 0)


def kernel(x, token_table, pos_table):
    idx2 = x.astype(jnp.int32).reshape(_B * _L // _HALF, _HALF)
    mesh = plsc.VectorSubcoreMesh(core_axis_name="c", subcore_axis_name="s")
    run = pl.kernel(
        _emb_body,
        out_type=jax.ShapeDtypeStruct((_B, _L, _D), jnp.float32),
        mesh=mesh,
        scratch_types=[
            pltpu.VMEM((64, _HALF), jnp.int32),
            pltpu.VMEM((_L, _D), jnp.float32),
            pltpu.VMEM((_L, _D), jnp.float32),
            pltpu.SemaphoreType.DMA,
            pltpu.SemaphoreType.DMA,
        ],
    )
    return run(idx2, token_table, pos_table)


# speedup vs baseline: 7.1007x; 1.6657x over previous
import jax
import jax.numpy as jnp
from jax import lax
from jax.experimental import pallas as pl
from jax.experimental.pallas import tpu as pltpu
from jax.experimental.pallas import tpu_sc as plsc

_B, _L, _V, _D = 1024, 200, 100000, 128
_NW = 32
_SEQ_PER_W = _B // _NW
_HALF = 100
_LANES = 16
_NBUF = 3


def _emb_body(idx_hbm, tok_hbm, pos_hbm, out_hbm, idx_v, pos_v,
              buf0, buf1, buf2, g0, g1, g2, s0, s1, s2):
    wid = lax.axis_index("s") * 2 + lax.axis_index("c")
    bufs = (buf0, buf1, buf2)
    gsems = (g0, g1, g2)
    ssems = (s0, s1, s2)

    pltpu.sync_copy(idx_hbm.at[pl.ds(64 * wid, 64)], idx_v)
    pltpu.sync_copy(pos_hbm, pos_v)

    def gather_descs(c):
        b = c % _NBUF
        return (
            (tok_hbm.at[idx_v.at[2 * c]], bufs[b].at[pl.ds(0, _HALF)], gsems[b]),
            (tok_hbm.at[idx_v.at[2 * c + 1]], bufs[b].at[pl.ds(_HALF, _HALF)],
             gsems[b]),
        )

    def scatter_desc(c):
        b = c % _NBUF
        return (bufs[b], out_hbm.at[_SEQ_PER_W * wid + c], ssems[b])

    def issue_gather(c):
        for d in gather_descs(c):
            pltpu.async_copy(*d)

    def wait_gather(c):
        for d in gather_descs(c):
            pltpu.make_async_copy(*d).wait()

    def add_pos(buf):
        def row(r, rc):
            for u in range(2):
                for j in range(_D // _LANES):
                    sl = pl.ds(_LANES * j, _LANES)
                    plsc.addupdate(buf.at[2 * r + u, sl], pos_v[2 * r + u, sl])
            return rc
        lax.fori_loop(0, _L // 2, row, 0)

    issue_gather(0)
    issue_gather(1)
    for c in range(_SEQ_PER_W):
        b = c % _NBUF
        wait_gather(c)
        add_pos(bufs[b])
        pltpu.async_copy(*scatter_desc(c))
        if c + 2 < _SEQ_PER_W:
            if c >= 1:
                pltpu.make_async_copy(*scatter_desc(c - 1)).wait()
            issue_gather(c + 2)
    for c in range(_SEQ_PER_W - _NBUF, _SEQ_PER_W):
        pltpu.make_async_copy(*scatter_desc(c)).wait()


def kernel(x, token_table, pos_table):
    idx2 = x.astype(jnp.int32).reshape(_B * _L // _HALF, _HALF)
    mesh = plsc.VectorSubcoreMesh(core_axis_name="c", subcore_axis_name="s")
    run = pl.kernel(
        _emb_body,
        out_type=jax.ShapeDtypeStruct((_B, _L, _D), jnp.float32),
        mesh=mesh,
        scratch_types=[
            pltpu.VMEM((64, _HALF), jnp.int32),
            pltpu.VMEM((_L, _D), jnp.float32),
            pltpu.VMEM((_L, _D), jnp.float32),
            pltpu.VMEM((_L, _D), jnp.float32),
            pltpu.VMEM((_L, _D), jnp.float32),
            pltpu.SemaphoreType.DMA,
            pltpu.SemaphoreType.DMA,
            pltpu.SemaphoreType.DMA,
            pltpu.SemaphoreType.DMA,
            pltpu.SemaphoreType.DMA,
            pltpu.SemaphoreType.DMA,
        ],
    )
    return run(idx2, token_table, pos_table)
